# baseline (speedup 1.0000x reference)
import functools

import jax
import jax.numpy as jnp
from jax import lax
from jax.experimental import pallas as pl
from jax.experimental.pallas import tpu as pltpu
from jax.experimental.pallas import tpu_sc as plsc

_V = 1000
_H = 128
_VP = 1024
_NC = 2
_NS = 16
_NV = 32
_HB = 1024
_VG = 16
_L = 16


def _tableT_body(w_ref, e_ref, b_ref, t_ref):
    t_ref[...] = (
        jnp.dot(w_ref[...], e_ref[...].T, preferred_element_type=jnp.float32)
        + b_ref[...]
    )


def _make_tableT(W, Ep, b2d):
    return pl.pallas_call(
        _tableT_body,
        out_shape=jax.ShapeDtypeStruct((_V, _VP), jnp.float32),
    )(W, Ep, b2d)


def _make_gather(bsz, seq):
    mesh = plsc.VectorSubcoreMesh(core_axis_name="c", subcore_axis_name="s")

    @functools.partial(
        pl.kernel,
        mesh=mesh,
        compiler_params=pltpu.CompilerParams(needs_layout_passes=False),
        out_type=jax.ShapeDtypeStruct((seq, _V, bsz), jnp.float32),
        scratch_types=[
            pltpu.VMEM((_NV, _VP), jnp.float32),
            pltpu.VMEM((seq, _HB), jnp.int32),
            pltpu.VMEM((1, _VG, _HB), jnp.float32),
            pltpu.VMEM((1, _VG, _HB), jnp.float32),
            pltpu.SemaphoreType.DMA,
            pltpu.SemaphoreType.DMA,
        ],
    )
    def gather(tableT_hbm, idsT_hbm, out_hbm, tsl, idsv, st0, st1, ws0, ws1):
        sid = lax.axis_index("s")
        wid = sid * _NC + lax.axis_index("c")
        vbase = lax.min(wid * _NV, _V - _NV)

        pltpu.sync_copy(tableT_hbm.at[pl.ds(vbase, _NV)], tsl)
        pltpu.sync_copy(idsT_hbm, idsv)

        stages = ((st0, ws0), (st1, ws1))
        n_g = _HB // _L

        def s_body(s, carry):
            for vg in range(_NV // _VG):
                st, ws = stages[vg % 2]

                def _drain(st=st, ws=ws):
                    pltpu.make_async_copy(
                        st,
                        out_hbm.at[pl.ds(s, 1), pl.ds(vbase, _VG), :],
                        ws,
                    ).wait()

                if vg >= 2:
                    _drain()
                else:
                    pl.when(s > 0)(_drain)

                @plsc.parallel_loop(0, n_g, unroll=8)
                def _(g, _vg=vg, _st=st):
                    cols = idsv[s, pl.ds(g * _L, _L)]
                    for v8 in range(_VG):
                        rows = jnp.full((_L,), _vg * _VG + v8, jnp.int32)
                        vals = plsc.load_gather(tsl, [rows, cols])
                        _st[0, v8, pl.ds(g * _L, _L)] = vals
                pltpu.async_copy(
                    st,
                    out_hbm.at[pl.ds(s, 1), pl.ds(vbase + vg * _VG, _VG), :],
                    ws,
                )
            return carry

        lax.fori_loop(0, seq, s_body, 0)
        for st, ws in stages:
            pltpu.make_async_copy(
                st, out_hbm.at[pl.ds(0, 1), pl.ds(vbase, _VG), :], ws
            ).wait()

    return gather


def kernel(input_ids, embed, W, b):
    bsz, seq = input_ids.shape
    Ep = jnp.pad(embed, ((0, _VP - _V), (0, 0)))
    tableT = _make_tableT(W, Ep, b.reshape(_V, 1))
    idsT = input_ids.astype(jnp.int32).T
    out_phys = _make_gather(bsz, seq)(tableT, idsT)
    return jnp.transpose(out_phys, (2, 0, 1))


# speedup vs baseline: 1.1891x; 1.1891x over previous
import functools

import jax
import jax.numpy as jnp
from jax import lax
from jax.experimental import pallas as pl
from jax.experimental.pallas import tpu as pltpu
from jax.experimental.pallas import tpu_sc as plsc

_V = 1000
_H = 128
_VP = 1024
_NC = 2
_NS = 16
_NV = 64
_HB = 512
_VG = 8
_L = 16


def _tableT_body(w_ref, e_ref, b_ref, t_ref):
    t_ref[...] = (
        jnp.dot(w_ref[...], e_ref[...].T, preferred_element_type=jnp.float32)
        + b_ref[...]
    )


def _make_tableT(W, Ep, b2d):
    return pl.pallas_call(
        _tableT_body,
        out_shape=jax.ShapeDtypeStruct((_V, _VP), jnp.float32),
    )(W, Ep, b2d)


def _make_gather(bsz, seq):
    mesh = plsc.VectorSubcoreMesh(core_axis_name="c", subcore_axis_name="s")

    @functools.partial(
        pl.kernel,
        mesh=mesh,
        compiler_params=pltpu.CompilerParams(needs_layout_passes=False),
        out_type=jax.ShapeDtypeStruct((seq, _V, bsz), jnp.float32),
        scratch_types=[
            pltpu.VMEM((_NV, _VP), jnp.float32),
            pltpu.VMEM((seq, _HB), jnp.int32),
            pltpu.VMEM((1, _VG, _HB), jnp.float32),
            pltpu.VMEM((1, _VG, _HB), jnp.float32),
            pltpu.SemaphoreType.DMA,
            pltpu.SemaphoreType.DMA,
        ],
    )
    def gather(tableT_hbm, idsT_hbm, out_hbm, tsl, idsv, st0, st1, ws0, ws1):
        sid = lax.axis_index("s")
        wid = sid * _NC + lax.axis_index("c")
        r = lax.div(wid, 2)
        half = lax.rem(wid, 2)
        vbase = lax.min(r * _NV, _V - _NV)
        bbase = half * _HB

        pltpu.sync_copy(tableT_hbm.at[pl.ds(vbase, _NV)], tsl)
        pltpu.sync_copy(idsT_hbm.at[:, pl.ds(bbase, _HB)], idsv)

        stages = ((st0, ws0), (st1, ws1))
        n_g = _HB // _L

        def s_body(s, carry):
            for vg in range(_NV // _VG):
                st, ws = stages[vg % 2]

                def _drain(st=st, ws=ws):
                    pltpu.make_async_copy(
                        st,
                        out_hbm.at[pl.ds(s, 1), pl.ds(vbase, _VG), pl.ds(bbase, _HB)],
                        ws,
                    ).wait()

                if vg >= 2:
                    _drain()
                else:
                    pl.when(s > 0)(_drain)

                @plsc.parallel_loop(0, n_g, unroll=8)
                def _(g, _vg=vg, _st=st):
                    cols = idsv[s, pl.ds(g * _L, _L)]
                    for v8 in range(_VG):
                        rows = jnp.full((_L,), _vg * _VG + v8, jnp.int32)
                        vals = plsc.load_gather(tsl, [rows, cols])
                        _st[0, v8, pl.ds(g * _L, _L)] = vals
                pltpu.async_copy(
                    st,
                    out_hbm.at[
                        pl.ds(s, 1), pl.ds(vbase + vg * _VG, _VG), pl.ds(bbase, _HB)
                    ],
                    ws,
                )
            return carry

        lax.fori_loop(0, seq, s_body, 0)
        for st, ws in stages:
            pltpu.make_async_copy(
                st, out_hbm.at[pl.ds(0, 1), pl.ds(vbase, _VG), pl.ds(bbase, _HB)], ws
            ).wait()

    return gather


def kernel(input_ids, embed, W, b):
    bsz, seq = input_ids.shape
    Ep = jnp.pad(embed, ((0, _VP - _V), (0, 0)))
    tableT = _make_tableT(W, Ep, b.reshape(_V, 1))
    idsT = input_ids.astype(jnp.int32).T
    out_phys = _make_gather(bsz, seq)(tableT, idsT)
    return jnp.transpose(out_phys, (2, 0, 1))
